# baseline (speedup 1.0000x reference)
import functools

import jax
import jax.numpy as jnp
from jax import lax
from jax.experimental import pallas as pl
from jax.experimental.pallas import tpu as pltpu
from jax.experimental.pallas import tpu_sc as plsc

N = 10000
E = 160000
D = 256
H = 256
O = 128

NC = 2
NS = 16
HALF = 128
K = 125
CPT = E // NS // K
CPH = CPT // 2
WT = 10
ZROWS = N // NS


def _segsum_sc(table2, gidx, didx, zrows):
  mesh = plsc.VectorSubcoreMesh(core_axis_name="c", subcore_axis_name="s")

  @functools.partial(
      pl.kernel,
      mesh=mesh,
      out_type=jax.ShapeDtypeStruct((NC, N, HALF), jnp.float32),
      scratch_types=[
          pltpu.VMEM((CPH, K), jnp.int32),
          pltpu.VMEM((CPH, K), jnp.int32),
          pltpu.VMEM((K, HALF), jnp.float32),
          pltpu.VMEM((K, HALF), jnp.float32),
          pltpu.VMEM_SHARED((N, HALF), jnp.float32),
          pltpu.SemaphoreType.DMA,
          pltpu.SemaphoreType.DMA,
      ],
  )
  def seg_kernel(table_hbm, gidx_hbm, didx_hbm, zrows_hbm, out_hbm,
                 src_v, dst_v, buf_a, buf_b, acc, sem_a, sem_b):
    c = lax.axis_index("c")
    s = lax.axis_index("s")

    pltpu.sync_copy(zrows_hbm, acc.at[pl.ds(s * ZROWS, ZROWS), :])
    plsc.subcore_barrier()

    for h in range(2):
      base = s * CPT + h * CPH
      pltpu.sync_copy(gidx_hbm.at[c, pl.ds(base, CPH), :], src_v)
      pltpu.sync_copy(didx_hbm.at[pl.ds(base, CPH), :], dst_v)

      pltpu.async_copy(table_hbm.at[src_v.at[0]], buf_a, sem_a)

      @pl.loop(0, CPH, step=2)
      def _chunks(j):
        pltpu.make_async_copy(table_hbm.at[src_v.at[j]], buf_a, sem_a).wait()
        pltpu.async_copy(table_hbm.at[src_v.at[j + 1]], buf_b, sem_b)
        pltpu.sync_copy(buf_a, acc.at[dst_v.at[j]], add=True)
        pltpu.make_async_copy(table_hbm.at[src_v.at[j + 1]], buf_b,
                              sem_b).wait()

        @pl.when(j + 2 < CPH)
        def _():
          pltpu.async_copy(table_hbm.at[src_v.at[j + 2]], buf_a, sem_a)

        pltpu.sync_copy(buf_b, acc.at[dst_v.at[j + 1]], add=True)

    plsc.subcore_barrier()

    @pl.when(s < WT)
    def _():
      rpt = N // WT
      pltpu.sync_copy(acc.at[pl.ds(s * rpt, rpt), :],
                      out_hbm.at[c, pl.ds(s * rpt, rpt), :])

  return seg_kernel(table2, gidx, didx, zrows)


def _msg_tc(x, W_nbr):
  R = 1000

  def body(x_ref, wn_ref, m_ref):
    mm = jnp.dot(x_ref[...], wn_ref[...], preferred_element_type=jnp.float32)
    m_ref[0] = mm[:, :HALF]
    m_ref[1] = mm[:, HALF:]

  return pl.pallas_call(
      body,
      grid=(N // R,),
      in_specs=[
          pl.BlockSpec((R, D), lambda i: (i, 0)),
          pl.BlockSpec((D, H), lambda i: (0, 0)),
      ],
      out_specs=pl.BlockSpec((NC, R, HALF), lambda i: (0, i, 0)),
      out_shape=jax.ShapeDtypeStruct((NC, N, HALF), jnp.float32),
  )(x, W_nbr)


def _mm_tc(x, W):
  R = 1000

  def body(x_ref, w_ref, r_ref):
    r_ref[...] = jnp.dot(x_ref[...], w_ref[...],
                         preferred_element_type=jnp.float32)

  return pl.pallas_call(
      body,
      grid=(N // R,),
      in_specs=[
          pl.BlockSpec((R, D), lambda i: (i, 0)),
          pl.BlockSpec((D, H), lambda i: (0, 0)),
      ],
      out_specs=pl.BlockSpec((R, H), lambda i: (i, 0)),
      out_shape=jax.ShapeDtypeStruct((N, H), jnp.float32),
  )(x, W)


def _conv2_tc(r1, agg1, b1, W_nbr2):
  R = 1000

  def body(r1_ref, agg_ref, b_ref, wn_ref, m_ref, h_ref):
    h1 = jnp.concatenate(
        [r1_ref[:, :HALF] + agg_ref[0], r1_ref[:, HALF:] + agg_ref[1]],
        axis=1) + b_ref[...]
    mm = jnp.dot(h1, wn_ref[...], preferred_element_type=jnp.float32)
    m_ref[0] = mm[:, :HALF]
    m_ref[1] = mm[:, HALF:]
    h_ref[...] = h1

  return pl.pallas_call(
      body,
      grid=(N // R,),
      in_specs=[
          pl.BlockSpec((R, H), lambda i: (i, 0)),
          pl.BlockSpec((NC, R, HALF), lambda i: (0, i, 0)),
          pl.BlockSpec((1, H), lambda i: (0, 0)),
          pl.BlockSpec((H, H), lambda i: (0, 0)),
      ],
      out_specs=[
          pl.BlockSpec((NC, R, HALF), lambda i: (0, i, 0)),
          pl.BlockSpec((R, H), lambda i: (i, 0)),
      ],
      out_shape=[
          jax.ShapeDtypeStruct((NC, N, HALF), jnp.float32),
          jax.ShapeDtypeStruct((N, H), jnp.float32),
      ],
  )(r1, agg1, b1.reshape(1, H), W_nbr2)


def _tail_tc(r2, agg2, b2, W_ih0T, b_ih0, b_hh0, W_ih1T, b_ih1, b_hh1,
             W_fc, b_fc):
  R = 1000

  def gates(gi, bhh):
    r = jax.nn.sigmoid(gi[:, 0:H] + bhh[:, 0:H])
    z = jax.nn.sigmoid(gi[:, H:2 * H] + bhh[:, H:2 * H])
    n = jnp.tanh(gi[:, 2 * H:3 * H] + r * bhh[:, 2 * H:3 * H])
    return (1.0 - z) * n

  def body(r2_ref, agg_ref, b2_ref, wih0_ref, bih0_ref, bhh0_ref,
           wih1_ref, bih1_ref, bhh1_ref, wfc_ref, bfc_ref, out_ref):
    h2 = jnp.concatenate(
        [r2_ref[:, :HALF] + agg_ref[0], r2_ref[:, HALF:] + agg_ref[1]],
        axis=1) + b2_ref[...]
    gi0 = jnp.dot(h2, wih0_ref[...],
                  preferred_element_type=jnp.float32) + bih0_ref[...]
    o1 = gates(gi0, bhh0_ref[...])
    gi1 = jnp.dot(o1, wih1_ref[...],
                  preferred_element_type=jnp.float32) + bih1_ref[...]
    o2 = gates(gi1, bhh1_ref[...])
    out_ref[...] = jnp.dot(
        o2, wfc_ref[...], preferred_element_type=jnp.float32) + bfc_ref[...]

  full = lambda shape: pl.BlockSpec(shape, lambda i: tuple(0 for _ in shape))
  return pl.pallas_call(
      body,
      grid=(N // R,),
      in_specs=[
          pl.BlockSpec((R, H), lambda i: (i, 0)),
          pl.BlockSpec((NC, R, HALF), lambda i: (0, i, 0)),
          full((1, H)),
          full((H, 3 * H)),
          full((1, 3 * H)),
          full((1, 3 * H)),
          full((H, 3 * H)),
          full((1, 3 * H)),
          full((1, 3 * H)),
          full((H, O)),
          full((1, O)),
      ],
      out_specs=pl.BlockSpec((R, O), lambda i: (i, 0)),
      out_shape=jax.ShapeDtypeStruct((N, O), jnp.float32),
  )(r2, agg2, b2.reshape(1, H), W_ih0T, b_ih0.reshape(1, 3 * H),
    b_hh0.reshape(1, 3 * H), W_ih1T, b_ih1.reshape(1, 3 * H),
    b_hh1.reshape(1, 3 * H), W_fc, b_fc.reshape(1, O))


def kernel(x, edge_index, W_root1, W_nbr1, b1, W_root2, W_nbr2, b2,
           W_ih0, W_hh0, b_ih0, b_hh0, W_ih1, W_hh1, b_ih1, b_hh1,
           W_fc, b_fc):
  src = edge_index[0]
  dst = edge_index[1]
  gidx = jnp.stack([src, N + src]).reshape(NC, E // K, K)
  didx = dst.reshape(E // K, K)
  zrows = jnp.zeros((ZROWS, HALF), jnp.float32)

  m1 = _msg_tc(x, W_nbr1)
  agg1 = _segsum_sc(m1.reshape(2 * N, HALF), gidx, didx, zrows)
  r1 = _mm_tc(x, W_root1)
  m2, h1 = _conv2_tc(r1, agg1, b1, W_nbr2)
  agg2 = _segsum_sc(m2.reshape(2 * N, HALF), gidx, didx, zrows)
  r2 = _mm_tc(h1, W_root2)
  return _tail_tc(r2, agg2, b2, W_ih0.T, b_ih0, b_hh0, W_ih1.T, b_ih1,
                  b_hh1, W_fc, b_fc)


# speedup vs baseline: 1.0052x; 1.0052x over previous
import functools

import jax
import jax.numpy as jnp
from jax import lax
from jax.experimental import pallas as pl
from jax.experimental.pallas import tpu as pltpu
from jax.experimental.pallas import tpu_sc as plsc

N = 10000
E = 160000
D = 256
H = 256
O = 128

NC = 2
NS = 16
HALF = 128
K = 125
CPT = E // NS // K
CPH = CPT // 2
WT = 10
ZROWS = N // NS


def _segsum_sc(table2, gidx, didx, zrows):
  mesh = plsc.VectorSubcoreMesh(core_axis_name="c", subcore_axis_name="s")

  @functools.partial(
      pl.kernel,
      mesh=mesh,
      out_type=jax.ShapeDtypeStruct((NC, N, HALF), jnp.float32),
      scratch_types=[
          pltpu.VMEM((CPH, K), jnp.int32),
          pltpu.VMEM((CPH, K), jnp.int32),
          pltpu.VMEM((K, HALF), jnp.float32),
          pltpu.VMEM((K, HALF), jnp.float32),
          pltpu.VMEM_SHARED((N, HALF), jnp.float32),
          pltpu.SemaphoreType.DMA,
          pltpu.SemaphoreType.DMA,
      ],
  )
  def seg_kernel(table_hbm, gidx_hbm, didx_hbm, zrows_hbm, out_hbm,
                 src_v, dst_v, buf_a, buf_b, acc, sem_a, sem_b):
    c = lax.axis_index("c")
    s = lax.axis_index("s")

    pltpu.sync_copy(zrows_hbm, acc.at[pl.ds(s * ZROWS, ZROWS), :])
    plsc.subcore_barrier()

    for h in range(2):
      base = s * CPT + h * CPH
      pltpu.sync_copy(gidx_hbm.at[c, pl.ds(base, CPH), :], src_v)
      pltpu.sync_copy(didx_hbm.at[pl.ds(base, CPH), :], dst_v)

      pltpu.async_copy(table_hbm.at[src_v.at[0]], buf_a, sem_a)

      @pl.loop(0, CPH, step=2)
      def _chunks(j):
        pltpu.make_async_copy(table_hbm.at[src_v.at[j]], buf_a, sem_a).wait()
        pltpu.async_copy(table_hbm.at[src_v.at[j + 1]], buf_b, sem_b)
        pltpu.sync_copy(buf_a, acc.at[dst_v.at[j]], add=True)
        pltpu.make_async_copy(table_hbm.at[src_v.at[j + 1]], buf_b,
                              sem_b).wait()

        @pl.when(j + 2 < CPH)
        def _():
          pltpu.async_copy(table_hbm.at[src_v.at[j + 2]], buf_a, sem_a)

        pltpu.sync_copy(buf_b, acc.at[dst_v.at[j + 1]], add=True)

    plsc.subcore_barrier()

    @pl.when(s < WT)
    def _():
      rpt = N // WT
      pltpu.sync_copy(acc.at[pl.ds(s * rpt, rpt), :],
                      out_hbm.at[c, pl.ds(s * rpt, rpt), :])

  return seg_kernel(table2, gidx, didx, zrows)


def _conv_mm_tc(x, W_nbr, W_root):
  R = 1000

  def body(x_ref, wn_ref, wr_ref, m_ref, r_ref):
    mm = jnp.dot(x_ref[...], wn_ref[...], preferred_element_type=jnp.float32)
    m_ref[0] = mm[:, :HALF]
    m_ref[1] = mm[:, HALF:]
    r_ref[...] = jnp.dot(x_ref[...], wr_ref[...],
                         preferred_element_type=jnp.float32)

  return pl.pallas_call(
      body,
      grid=(N // R,),
      in_specs=[
          pl.BlockSpec((R, D), lambda i: (i, 0)),
          pl.BlockSpec((D, H), lambda i: (0, 0)),
          pl.BlockSpec((D, H), lambda i: (0, 0)),
      ],
      out_specs=[
          pl.BlockSpec((NC, R, HALF), lambda i: (0, i, 0)),
          pl.BlockSpec((R, H), lambda i: (i, 0)),
      ],
      out_shape=[
          jax.ShapeDtypeStruct((NC, N, HALF), jnp.float32),
          jax.ShapeDtypeStruct((N, H), jnp.float32),
      ],
  )(x, W_nbr, W_root)


def _conv2_tc(r1, agg1, b1, W_nbr2, W_root2):
  R = 1000

  def body(r1_ref, agg_ref, b_ref, wn_ref, wr_ref, m_ref, r_ref):
    h1 = jnp.concatenate(
        [r1_ref[:, :HALF] + agg_ref[0], r1_ref[:, HALF:] + agg_ref[1]],
        axis=1) + b_ref[...]
    mm = jnp.dot(h1, wn_ref[...], preferred_element_type=jnp.float32)
    m_ref[0] = mm[:, :HALF]
    m_ref[1] = mm[:, HALF:]
    r_ref[...] = jnp.dot(h1, wr_ref[...], preferred_element_type=jnp.float32)

  return pl.pallas_call(
      body,
      grid=(N // R,),
      in_specs=[
          pl.BlockSpec((R, H), lambda i: (i, 0)),
          pl.BlockSpec((NC, R, HALF), lambda i: (0, i, 0)),
          pl.BlockSpec((1, H), lambda i: (0, 0)),
          pl.BlockSpec((H, H), lambda i: (0, 0)),
          pl.BlockSpec((H, H), lambda i: (0, 0)),
      ],
      out_specs=[
          pl.BlockSpec((NC, R, HALF), lambda i: (0, i, 0)),
          pl.BlockSpec((R, H), lambda i: (i, 0)),
      ],
      out_shape=[
          jax.ShapeDtypeStruct((NC, N, HALF), jnp.float32),
          jax.ShapeDtypeStruct((N, H), jnp.float32),
      ],
  )(r1, agg1, b1.reshape(1, H), W_nbr2, W_root2)


def _tail_tc(r2, agg2, b2, W_ih0T, b_ih0, b_hh0, W_ih1T, b_ih1, b_hh1,
             W_fc, b_fc):
  R = 1000

  def gates(gi, bhh):
    r = jax.nn.sigmoid(gi[:, 0:H] + bhh[:, 0:H])
    z = jax.nn.sigmoid(gi[:, H:2 * H] + bhh[:, H:2 * H])
    n = jnp.tanh(gi[:, 2 * H:3 * H] + r * bhh[:, 2 * H:3 * H])
    return (1.0 - z) * n

  def body(r2_ref, agg_ref, b2_ref, wih0_ref, bih0_ref, bhh0_ref,
           wih1_ref, bih1_ref, bhh1_ref, wfc_ref, bfc_ref, out_ref):
    h2 = jnp.concatenate(
        [r2_ref[:, :HALF] + agg_ref[0], r2_ref[:, HALF:] + agg_ref[1]],
        axis=1) + b2_ref[...]
    gi0 = jnp.dot(h2, wih0_ref[...],
                  preferred_element_type=jnp.float32) + bih0_ref[...]
    o1 = gates(gi0, bhh0_ref[...])
    gi1 = jnp.dot(o1, wih1_ref[...],
                  preferred_element_type=jnp.float32) + bih1_ref[...]
    o2 = gates(gi1, bhh1_ref[...])
    out_ref[...] = jnp.dot(
        o2, wfc_ref[...], preferred_element_type=jnp.float32) + bfc_ref[...]

  full = lambda shape: pl.BlockSpec(shape, lambda i: tuple(0 for _ in shape))
  return pl.pallas_call(
      body,
      grid=(N // R,),
      in_specs=[
          pl.BlockSpec((R, H), lambda i: (i, 0)),
          pl.BlockSpec((NC, R, HALF), lambda i: (0, i, 0)),
          full((1, H)),
          full((H, 3 * H)),
          full((1, 3 * H)),
          full((1, 3 * H)),
          full((H, 3 * H)),
          full((1, 3 * H)),
          full((1, 3 * H)),
          full((H, O)),
          full((1, O)),
      ],
      out_specs=pl.BlockSpec((R, O), lambda i: (i, 0)),
      out_shape=jax.ShapeDtypeStruct((N, O), jnp.float32),
  )(r2, agg2, b2.reshape(1, H), W_ih0T, b_ih0.reshape(1, 3 * H),
    b_hh0.reshape(1, 3 * H), W_ih1T, b_ih1.reshape(1, 3 * H),
    b_hh1.reshape(1, 3 * H), W_fc, b_fc.reshape(1, O))


def kernel(x, edge_index, W_root1, W_nbr1, b1, W_root2, W_nbr2, b2,
           W_ih0, W_hh0, b_ih0, b_hh0, W_ih1, W_hh1, b_ih1, b_hh1,
           W_fc, b_fc):
  src = edge_index[0]
  dst = edge_index[1]
  gidx = jnp.stack([src, N + src]).reshape(NC, E // K, K)
  didx = dst.reshape(E // K, K)
  zrows = jnp.zeros((ZROWS, HALF), jnp.float32)

  m1, r1 = _conv_mm_tc(x, W_nbr1, W_root1)
  agg1 = _segsum_sc(m1.reshape(2 * N, HALF), gidx, didx, zrows)
  m2, r2 = _conv2_tc(r1, agg1, b1, W_nbr2, W_root2)
  agg2 = _segsum_sc(m2.reshape(2 * N, HALF), gidx, didx, zrows)
  return _tail_tc(r2, agg2, b2, W_ih0.T, b_ih0, b_hh0, W_ih1.T, b_ih1,
                  b_hh1, W_fc, b_fc)


# speedup vs baseline: 1.0245x; 1.0191x over previous
import functools

import jax
import jax.numpy as jnp
from jax import lax
from jax.experimental import pallas as pl
from jax.experimental.pallas import tpu as pltpu
from jax.experimental.pallas import tpu_sc as plsc

N = 10000
E = 160000
D = 256
H = 256
O = 128

NC = 2
NS = 16
HALF = 128
K = 125
CPT = E // NS // K
CPH = CPT // 2
WT = 10
ZROWS = N // NS


def _segsum_sc(table2, gidx, didx, zrows):
  mesh = plsc.VectorSubcoreMesh(core_axis_name="c", subcore_axis_name="s")

  @functools.partial(
      pl.kernel,
      mesh=mesh,
      out_type=jax.ShapeDtypeStruct((NC, N, HALF), jnp.float32),
      scratch_types=[
          pltpu.VMEM((CPT, K), jnp.int32),
          pltpu.VMEM((CPH, K), jnp.int32),
          pltpu.VMEM((K, HALF), jnp.float32),
          pltpu.VMEM((K, HALF), jnp.float32),
          pltpu.VMEM_SHARED((N, HALF), jnp.float32),
          pltpu.SemaphoreType.DMA,
          pltpu.SemaphoreType.DMA,
      ],
  )
  def seg_kernel(table_hbm, gidx_hbm, didx_hbm, zrows_hbm, out_hbm,
                 src_v, dst_v, buf_a, buf_b, acc, sem_a, sem_b):
    c = lax.axis_index("c")
    s = lax.axis_index("s")

    pltpu.sync_copy(gidx_hbm.at[c, pl.ds(s * CPT, CPT), :], src_v)
    pltpu.sync_copy(didx_hbm.at[pl.ds(s * CPT, CPH), :], dst_v)
    pltpu.async_copy(table_hbm.at[src_v.at[0]], buf_a, sem_a)
    pltpu.sync_copy(zrows_hbm, acc.at[pl.ds(s * ZROWS, ZROWS), :])
    plsc.subcore_barrier()

    @pl.loop(0, CPT, step=2)
    def _chunks(j):
      @pl.when(j == CPH)
      def _():
        pltpu.sync_copy(didx_hbm.at[pl.ds(s * CPT + CPH, CPH), :], dst_v)

      jj = lax.select(j >= CPH, j - CPH, j)
      pltpu.make_async_copy(table_hbm.at[src_v.at[j]], buf_a, sem_a).wait()
      pltpu.async_copy(table_hbm.at[src_v.at[j + 1]], buf_b, sem_b)
      pltpu.sync_copy(buf_a, acc.at[dst_v.at[jj]], add=True)
      pltpu.make_async_copy(table_hbm.at[src_v.at[j + 1]], buf_b,
                            sem_b).wait()

      @pl.when(j + 2 < CPT)
      def _():
        pltpu.async_copy(table_hbm.at[src_v.at[j + 2]], buf_a, sem_a)

      pltpu.sync_copy(buf_b, acc.at[dst_v.at[jj + 1]], add=True)

    plsc.subcore_barrier()

    @pl.when(s < WT)
    def _():
      rpt = N // WT
      pltpu.sync_copy(acc.at[pl.ds(s * rpt, rpt), :],
                      out_hbm.at[c, pl.ds(s * rpt, rpt), :])

  return seg_kernel(table2, gidx, didx, zrows)


def _conv_mm_tc(x, W_nbr, W_root):
  R = 1000

  def body(x_ref, wn_ref, wr_ref, m_ref, r_ref):
    mm = jnp.dot(x_ref[...], wn_ref[...], preferred_element_type=jnp.float32)
    m_ref[0] = mm[:, :HALF]
    m_ref[1] = mm[:, HALF:]
    r_ref[...] = jnp.dot(x_ref[...], wr_ref[...],
                         preferred_element_type=jnp.float32)

  return pl.pallas_call(
      body,
      grid=(N // R,),
      in_specs=[
          pl.BlockSpec((R, D), lambda i: (i, 0)),
          pl.BlockSpec((D, H), lambda i: (0, 0)),
          pl.BlockSpec((D, H), lambda i: (0, 0)),
      ],
      out_specs=[
          pl.BlockSpec((NC, R, HALF), lambda i: (0, i, 0)),
          pl.BlockSpec((R, H), lambda i: (i, 0)),
      ],
      out_shape=[
          jax.ShapeDtypeStruct((NC, N, HALF), jnp.float32),
          jax.ShapeDtypeStruct((N, H), jnp.float32),
      ],
  )(x, W_nbr, W_root)


def _conv2_tc(r1, agg1, b1, W_nbr2, W_root2):
  R = 1000

  def body(r1_ref, agg_ref, b_ref, wn_ref, wr_ref, m_ref, r_ref):
    h1 = jnp.concatenate(
        [r1_ref[:, :HALF] + agg_ref[0], r1_ref[:, HALF:] + agg_ref[1]],
        axis=1) + b_ref[...]
    mm = jnp.dot(h1, wn_ref[...], preferred_element_type=jnp.float32)
    m_ref[0] = mm[:, :HALF]
    m_ref[1] = mm[:, HALF:]
    r_ref[...] = jnp.dot(h1, wr_ref[...], preferred_element_type=jnp.float32)

  return pl.pallas_call(
      body,
      grid=(N // R,),
      in_specs=[
          pl.BlockSpec((R, H), lambda i: (i, 0)),
          pl.BlockSpec((NC, R, HALF), lambda i: (0, i, 0)),
          pl.BlockSpec((1, H), lambda i: (0, 0)),
          pl.BlockSpec((H, H), lambda i: (0, 0)),
          pl.BlockSpec((H, H), lambda i: (0, 0)),
      ],
      out_specs=[
          pl.BlockSpec((NC, R, HALF), lambda i: (0, i, 0)),
          pl.BlockSpec((R, H), lambda i: (i, 0)),
      ],
      out_shape=[
          jax.ShapeDtypeStruct((NC, N, HALF), jnp.float32),
          jax.ShapeDtypeStruct((N, H), jnp.float32),
      ],
  )(r1, agg1, b1.reshape(1, H), W_nbr2, W_root2)


def _tail_tc(r2, agg2, b2, W_ih0T, b_ih0, b_hh0, W_ih1T, b_ih1, b_hh1,
             W_fc, b_fc):
  R = 1000

  def gates(gi, bhh):
    r = jax.nn.sigmoid(gi[:, 0:H] + bhh[:, 0:H])
    z = jax.nn.sigmoid(gi[:, H:2 * H] + bhh[:, H:2 * H])
    n = jnp.tanh(gi[:, 2 * H:3 * H] + r * bhh[:, 2 * H:3 * H])
    return (1.0 - z) * n

  def body(r2_ref, agg_ref, b2_ref, wih0_ref, bih0_ref, bhh0_ref,
           wih1_ref, bih1_ref, bhh1_ref, wfc_ref, bfc_ref, out_ref):
    h2 = jnp.concatenate(
        [r2_ref[:, :HALF] + agg_ref[0], r2_ref[:, HALF:] + agg_ref[1]],
        axis=1) + b2_ref[...]
    gi0 = jnp.dot(h2, wih0_ref[...],
                  preferred_element_type=jnp.float32) + bih0_ref[...]
    o1 = gates(gi0, bhh0_ref[...])
    gi1 = jnp.dot(o1, wih1_ref[...],
                  preferred_element_type=jnp.float32) + bih1_ref[...]
    o2 = gates(gi1, bhh1_ref[...])
    out_ref[...] = jnp.dot(
        o2, wfc_ref[...], preferred_element_type=jnp.float32) + bfc_ref[...]

  full = lambda shape: pl.BlockSpec(shape, lambda i: tuple(0 for _ in shape))
  return pl.pallas_call(
      body,
      grid=(N // R,),
      in_specs=[
          pl.BlockSpec((R, H), lambda i: (i, 0)),
          pl.BlockSpec((NC, R, HALF), lambda i: (0, i, 0)),
          full((1, H)),
          full((H, 3 * H)),
          full((1, 3 * H)),
          full((1, 3 * H)),
          full((H, 3 * H)),
          full((1, 3 * H)),
          full((1, 3 * H)),
          full((H, O)),
          full((1, O)),
      ],
      out_specs=pl.BlockSpec((R, O), lambda i: (i, 0)),
      out_shape=jax.ShapeDtypeStruct((N, O), jnp.float32),
  )(r2, agg2, b2.reshape(1, H), W_ih0T, b_ih0.reshape(1, 3 * H),
    b_hh0.reshape(1, 3 * H), W_ih1T, b_ih1.reshape(1, 3 * H),
    b_hh1.reshape(1, 3 * H), W_fc, b_fc.reshape(1, O))


def kernel(x, edge_index, W_root1, W_nbr1, b1, W_root2, W_nbr2, b2,
           W_ih0, W_hh0, b_ih0, b_hh0, W_ih1, W_hh1, b_ih1, b_hh1,
           W_fc, b_fc):
  src = edge_index[0]
  dst = edge_index[1]
  gidx = jnp.stack([src, N + src]).reshape(NC, E // K, K)
  didx = dst.reshape(E // K, K)
  zrows = jnp.zeros((ZROWS, HALF), jnp.float32)

  m1, r1 = _conv_mm_tc(x, W_nbr1, W_root1)
  agg1 = _segsum_sc(m1.reshape(2 * N, HALF), gidx, didx, zrows)
  m2, r2 = _conv2_tc(r1, agg1, b1, W_nbr2, W_root2)
  agg2 = _segsum_sc(m2.reshape(2 * N, HALF), gidx, didx, zrows)
  return _tail_tc(r2, agg2, b2, W_ih0.T, b_ih0, b_hh0, W_ih1.T, b_ih1,
                  b_hh1, W_fc, b_fc)


# speedup vs baseline: 1.0251x; 1.0006x over previous
import functools

import jax
import jax.numpy as jnp
from jax import lax
from jax.experimental import pallas as pl
from jax.experimental.pallas import tpu as pltpu
from jax.experimental.pallas import tpu_sc as plsc

N = 10000
E = 160000
D = 256
H = 256
O = 128

NC = 2
NS = 16
HALF = 128
K = 125
CPT = E // NS // K
CPH = CPT // 2
WT = 10
ZROWS = N // NS


def _segsum_sc(table2, gidx, didx, zrows):
  mesh = plsc.VectorSubcoreMesh(core_axis_name="c", subcore_axis_name="s")

  @functools.partial(
      pl.kernel,
      mesh=mesh,
      out_type=jax.ShapeDtypeStruct((NC, N, HALF), jnp.float32),
      scratch_types=[
          pltpu.VMEM((CPT, K), jnp.int32),
          pltpu.VMEM((CPH, K), jnp.int32),
          pltpu.VMEM((K, HALF), jnp.float32),
          pltpu.VMEM((K, HALF), jnp.float32),
          pltpu.VMEM_SHARED((N, HALF), jnp.float32),
          pltpu.SemaphoreType.DMA,
          pltpu.SemaphoreType.DMA,
      ],
  )
  def seg_kernel(table_hbm, gidx_hbm, didx_hbm, zrows_hbm, out_hbm,
                 src_v, dst_v, buf_a, buf_b, acc, sem_a, sem_b):
    c = lax.axis_index("c")
    s = lax.axis_index("s")

    pltpu.sync_copy(gidx_hbm.at[c, pl.ds(s * CPT, CPT), :], src_v)
    pltpu.sync_copy(didx_hbm.at[pl.ds(s * CPT, CPH), :], dst_v)
    pltpu.async_copy(table_hbm.at[src_v.at[0]], buf_a, sem_a)
    pltpu.sync_copy(zrows_hbm, acc.at[pl.ds(s * ZROWS, ZROWS), :])
    plsc.subcore_barrier()

    @pl.loop(0, CPT, step=2)
    def _chunks(j):
      @pl.when(j == CPH)
      def _():
        pltpu.sync_copy(didx_hbm.at[pl.ds(s * CPT + CPH, CPH), :], dst_v)

      jj = lax.select(j >= CPH, j - CPH, j)
      pltpu.make_async_copy(table_hbm.at[src_v.at[j]], buf_a, sem_a).wait()
      pltpu.async_copy(table_hbm.at[src_v.at[j + 1]], buf_b, sem_b)
      pltpu.sync_copy(buf_a, acc.at[dst_v.at[jj]], add=True)
      pltpu.make_async_copy(table_hbm.at[src_v.at[j + 1]], buf_b,
                            sem_b).wait()

      @pl.when(j + 2 < CPT)
      def _():
        pltpu.async_copy(table_hbm.at[src_v.at[j + 2]], buf_a, sem_a)

      pltpu.sync_copy(buf_b, acc.at[dst_v.at[jj + 1]], add=True)

    plsc.subcore_barrier()

    @pl.when(s < NS - 1)
    def _():
      pltpu.sync_copy(acc.at[pl.ds(s * 632, 632), :],
                      out_hbm.at[c, pl.ds(s * 632, 632), :])

    @pl.when(s == NS - 1)
    def _():
      pltpu.sync_copy(acc.at[pl.ds(632 * (NS - 1), N - 632 * (NS - 1)), :],
                      out_hbm.at[c, pl.ds(632 * (NS - 1),
                                          N - 632 * (NS - 1)), :])

  return seg_kernel(table2, gidx, didx, zrows)


def _conv_mm_tc(x, W_nbr, W_root):
  R = 1000

  def body(x_ref, wn_ref, wr_ref, m_ref, r_ref):
    mm = jnp.dot(x_ref[...], wn_ref[...], preferred_element_type=jnp.float32)
    m_ref[0] = mm[:, :HALF]
    m_ref[1] = mm[:, HALF:]
    r_ref[...] = jnp.dot(x_ref[...], wr_ref[...],
                         preferred_element_type=jnp.float32)

  return pl.pallas_call(
      body,
      grid=(N // R,),
      in_specs=[
          pl.BlockSpec((R, D), lambda i: (i, 0)),
          pl.BlockSpec((D, H), lambda i: (0, 0)),
          pl.BlockSpec((D, H), lambda i: (0, 0)),
      ],
      out_specs=[
          pl.BlockSpec((NC, R, HALF), lambda i: (0, i, 0)),
          pl.BlockSpec((R, H), lambda i: (i, 0)),
      ],
      out_shape=[
          jax.ShapeDtypeStruct((NC, N, HALF), jnp.float32),
          jax.ShapeDtypeStruct((N, H), jnp.float32),
      ],
  )(x, W_nbr, W_root)


def _conv2_tc(r1, agg1, b1, W_nbr2, W_root2):
  R = 1000

  def body(r1_ref, agg_ref, b_ref, wn_ref, wr_ref, m_ref, r_ref):
    h1 = jnp.concatenate(
        [r1_ref[:, :HALF] + agg_ref[0], r1_ref[:, HALF:] + agg_ref[1]],
        axis=1) + b_ref[...]
    mm = jnp.dot(h1, wn_ref[...], preferred_element_type=jnp.float32)
    m_ref[0] = mm[:, :HALF]
    m_ref[1] = mm[:, HALF:]
    r_ref[...] = jnp.dot(h1, wr_ref[...], preferred_element_type=jnp.float32)

  return pl.pallas_call(
      body,
      grid=(N // R,),
      in_specs=[
          pl.BlockSpec((R, H), lambda i: (i, 0)),
          pl.BlockSpec((NC, R, HALF), lambda i: (0, i, 0)),
          pl.BlockSpec((1, H), lambda i: (0, 0)),
          pl.BlockSpec((H, H), lambda i: (0, 0)),
          pl.BlockSpec((H, H), lambda i: (0, 0)),
      ],
      out_specs=[
          pl.BlockSpec((NC, R, HALF), lambda i: (0, i, 0)),
          pl.BlockSpec((R, H), lambda i: (i, 0)),
      ],
      out_shape=[
          jax.ShapeDtypeStruct((NC, N, HALF), jnp.float32),
          jax.ShapeDtypeStruct((N, H), jnp.float32),
      ],
  )(r1, agg1, b1.reshape(1, H), W_nbr2, W_root2)


def _tail_tc(r2, agg2, b2, W_ih0T, b_ih0, b_hh0, W_ih1T, b_ih1, b_hh1,
             W_fc, b_fc):
  R = 1000

  def gates(gi, bhh):
    r = jax.nn.sigmoid(gi[:, 0:H] + bhh[:, 0:H])
    z = jax.nn.sigmoid(gi[:, H:2 * H] + bhh[:, H:2 * H])
    n = jnp.tanh(gi[:, 2 * H:3 * H] + r * bhh[:, 2 * H:3 * H])
    return (1.0 - z) * n

  def body(r2_ref, agg_ref, b2_ref, wih0_ref, bih0_ref, bhh0_ref,
           wih1_ref, bih1_ref, bhh1_ref, wfc_ref, bfc_ref, out_ref):
    h2 = jnp.concatenate(
        [r2_ref[:, :HALF] + agg_ref[0], r2_ref[:, HALF:] + agg_ref[1]],
        axis=1) + b2_ref[...]
    gi0 = jnp.dot(h2, wih0_ref[...],
                  preferred_element_type=jnp.float32) + bih0_ref[...]
    o1 = gates(gi0, bhh0_ref[...])
    gi1 = jnp.dot(o1, wih1_ref[...],
                  preferred_element_type=jnp.float32) + bih1_ref[...]
    o2 = gates(gi1, bhh1_ref[...])
    out_ref[...] = jnp.dot(
        o2, wfc_ref[...], preferred_element_type=jnp.float32) + bfc_ref[...]

  full = lambda shape: pl.BlockSpec(shape, lambda i: tuple(0 for _ in shape))
  return pl.pallas_call(
      body,
      grid=(N // R,),
      in_specs=[
          pl.BlockSpec((R, H), lambda i: (i, 0)),
          pl.BlockSpec((NC, R, HALF), lambda i: (0, i, 0)),
          full((1, H)),
          full((H, 3 * H)),
          full((1, 3 * H)),
          full((1, 3 * H)),
          full((H, 3 * H)),
          full((1, 3 * H)),
          full((1, 3 * H)),
          full((H, O)),
          full((1, O)),
      ],
      out_specs=pl.BlockSpec((R, O), lambda i: (i, 0)),
      out_shape=jax.ShapeDtypeStruct((N, O), jnp.float32),
  )(r2, agg2, b2.reshape(1, H), W_ih0T, b_ih0.reshape(1, 3 * H),
    b_hh0.reshape(1, 3 * H), W_ih1T, b_ih1.reshape(1, 3 * H),
    b_hh1.reshape(1, 3 * H), W_fc, b_fc.reshape(1, O))


def kernel(x, edge_index, W_root1, W_nbr1, b1, W_root2, W_nbr2, b2,
           W_ih0, W_hh0, b_ih0, b_hh0, W_ih1, W_hh1, b_ih1, b_hh1,
           W_fc, b_fc):
  src = edge_index[0]
  dst = edge_index[1]
  gidx = jnp.stack([src, N + src]).reshape(NC, E // K, K)
  didx = dst.reshape(E // K, K)
  zrows = jnp.zeros((ZROWS, HALF), jnp.float32)

  m1, r1 = _conv_mm_tc(x, W_nbr1, W_root1)
  agg1 = _segsum_sc(m1.reshape(2 * N, HALF), gidx, didx, zrows)
  m2, r2 = _conv2_tc(r1, agg1, b1, W_nbr2, W_root2)
  agg2 = _segsum_sc(m2.reshape(2 * N, HALF), gidx, didx, zrows)
  return _tail_tc(r2, agg2, b2, W_ih0.T, b_ih0, b_hh0, W_ih1.T, b_ih1,
                  b_hh1, W_fc, b_fc)
